# Initial kernel scaffold; baseline (speedup 1.0000x reference)
import jax
import jax.numpy as jnp
from jax.experimental import pallas as pl


def kernel(x, edge_index, W_l, W_r, b_l):
    raise NotImplementedError("write your pallas kernel here")


# speedup vs baseline: 3.9254x; 3.9254x over previous
import jax
import jax.numpy as jnp
from jax import lax
from jax.experimental import pallas as pl
from jax.experimental.pallas import tpu as pltpu
from jax.experimental.pallas import tpu_sc as plsc

N = 10000
E = 320000
D = 128
DH = D // 2

NC = 2
NS = 16
CPT = E // NS
CHUNK = 80
NCHUNK = CPT // CHUNK
NP = 10240
RPT = NP // NS
ZR = 128


def _sc_body(x2_hbm, src_hbm, dst_hbm, out_s, out_c,
             sidx, didx, rows, ones16, zrows, z16, acc, cnt):
    c = lax.axis_index("c")
    s = lax.axis_index("s")
    ebase = s * CPT
    rbase = s * RPT

    zero16 = jnp.zeros((16,), jnp.float32)
    one16 = jnp.ones((16,), jnp.float32)

    def fill_z(i, carry):
        for k in range(DH // 16):
            zrows[i, pl.ds(k * 16, 16)] = zero16
        z16[i, :] = zero16
        return carry

    lax.fori_loop(0, ZR, fill_z, 0)

    def fill_one(i, carry):
        ones16[i, :] = one16
        return carry

    lax.fori_loop(0, CHUNK, fill_one, 0)

    def zcopy(k, carry):
        r0 = rbase + k * ZR
        pltpu.sync_copy(zrows, acc.at[pl.ds(r0, ZR)])
        pltpu.sync_copy(z16, cnt.at[pl.ds(r0, ZR)])
        return carry

    lax.fori_loop(0, RPT // ZR, zcopy, 0)
    plsc.subcore_barrier()

    my_x = x2_hbm.at[c]

    def edge(j, carry):
        off = ebase + j * CHUNK
        pltpu.sync_copy(src_hbm.at[pl.ds(off, CHUNK)], sidx)
        pltpu.sync_copy(dst_hbm.at[pl.ds(off, CHUNK)], didx.at[0])
        pltpu.sync_copy(my_x.at[sidx], rows)
        pltpu.sync_copy(rows, acc.at[didx.at[0]], add=True)

        @pl.when(c == 0)
        def _():
            pltpu.sync_copy(ones16, cnt.at[didx.at[0]], add=True)

        return carry

    lax.fori_loop(0, NCHUNK, edge, 0)
    plsc.subcore_barrier()

    def out_copy(k, carry):
        r0 = rbase + k * ZR
        pltpu.sync_copy(acc.at[pl.ds(r0, ZR)], zrows)
        pltpu.sync_copy(zrows, out_s.at[c, pl.ds(r0, ZR)])

        @pl.when(c == 0)
        def _():
            pltpu.sync_copy(cnt.at[pl.ds(r0, ZR)], z16)
            pltpu.sync_copy(z16, out_c.at[pl.ds(r0, ZR)])

        return carry

    lax.fori_loop(0, RPT // ZR, out_copy, 0)


_sc_segment_sum = pl.kernel(
    _sc_body,
    out_type=(
        jax.ShapeDtypeStruct((NC, NP, DH), jnp.float32),
        jax.ShapeDtypeStruct((NP, 16), jnp.float32),
    ),
    mesh=plsc.VectorSubcoreMesh(
        core_axis_name="c", subcore_axis_name="s",
        num_cores=NC, num_subcores=NS),
    compiler_params=pltpu.CompilerParams(use_tc_tiling_on_sc=False),
    scratch_types=[
        pltpu.VMEM((CHUNK,), jnp.int32),
        pltpu.VMEM((1, CHUNK), jnp.int32),
        pltpu.VMEM((CHUNK, DH), jnp.float32),
        pltpu.VMEM((CHUNK, 16), jnp.float32),
        pltpu.VMEM((ZR, DH), jnp.float32),
        pltpu.VMEM((ZR, 16), jnp.float32),
        pltpu.VMEM_SHARED((NP, DH), jnp.float32),
        pltpu.VMEM_SHARED((NP, 16), jnp.float32),
    ],
)


R = 1000


def _tc_body(s_ref, c_ref, x_ref, wl_ref, wr_ref, b_ref, o_ref):
    ssum = jnp.concatenate([s_ref[0], s_ref[1]], axis=-1)
    cnt = c_ref[...][:, :1]
    mean = ssum / jnp.clip(cnt, 1.0, None)
    dn = (((1,), (1,)), ((), ()))
    out = (lax.dot_general(mean, wl_ref[...], dn,
                           preferred_element_type=jnp.float32)
           + lax.dot_general(x_ref[...], wr_ref[...], dn,
                             preferred_element_type=jnp.float32)
           + b_ref[...])
    nrm = jnp.sqrt(jnp.sum(out * out, axis=-1, keepdims=True))
    o_ref[...] = out / jnp.maximum(nrm, 1e-12)


def _tc_combine(ps, pc, x, W_l, W_r, b2):
    return pl.pallas_call(
        _tc_body,
        grid=(N // R,),
        in_specs=[
            pl.BlockSpec((NC, R, DH), lambda i: (0, i, 0)),
            pl.BlockSpec((R, 16), lambda i: (i, 0)),
            pl.BlockSpec((R, D), lambda i: (i, 0)),
            pl.BlockSpec((D, D), lambda i: (0, 0)),
            pl.BlockSpec((D, D), lambda i: (0, 0)),
            pl.BlockSpec((1, D), lambda i: (0, 0)),
        ],
        out_specs=pl.BlockSpec((R, D), lambda i: (i, 0)),
        out_shape=jax.ShapeDtypeStruct((N, D), jnp.float32),
    )(ps, pc, x, W_l, W_r, b2)


@jax.jit
def kernel(x, edge_index, W_l, W_r, b_l):
    src = edge_index[0]
    dst = edge_index[1]
    x2 = jnp.stack([x[:, :DH], x[:, DH:]])
    ps, pc = _sc_segment_sum(x2, src, dst)
    return _tc_combine(ps, pc, x, W_l, W_r, b_l.reshape(1, D))


# speedup vs baseline: 10.4158x; 2.6535x over previous
import jax
import jax.numpy as jnp
from jax import lax
from jax.experimental import pallas as pl
from jax.experimental.pallas import tpu as pltpu
from jax.experimental.pallas import tpu_sc as plsc

N = 10000
E = 320000
D = 128
DH = D // 2

NC = 2
NS = 16
CPT = E // NS
CHUNK = 80
NCHUNK = CPT // CHUNK
NP = 10240
RPT = NP // NS
ZR = 128


def _sc_body(x2_hbm, src_hbm, dst_hbm, out_s, out_c,
             sidx_all, didx_all, rows0, rows1, ones16, zrows, z16,
             isem, gsem0, gsem1, csem, acc, cnt):
    c = lax.axis_index("c")
    s = lax.axis_index("s")
    jbase = s * NCHUNK
    rbase = s * RPT

    zero16 = jnp.zeros((16,), jnp.float32)
    one16 = jnp.ones((16,), jnp.float32)

    idx_in = pltpu.async_copy(src_hbm.at[pl.ds(jbase, NCHUNK)], sidx_all, isem)
    idx_in2 = pltpu.async_copy(dst_hbm.at[pl.ds(jbase, NCHUNK)], didx_all, isem)

    def fill_z(i, carry):
        for k in range(DH // 16):
            zrows[i, pl.ds(k * 16, 16)] = zero16
        z16[i, :] = zero16
        return carry

    lax.fori_loop(0, ZR, fill_z, 0)

    def fill_one(i, carry):
        ones16[i, :] = one16
        return carry

    lax.fori_loop(0, CHUNK, fill_one, 0)

    def zcopy(k, carry):
        r0 = rbase + k * ZR
        pltpu.sync_copy(zrows, acc.at[pl.ds(r0, ZR)])
        pltpu.sync_copy(z16, cnt.at[pl.ds(r0, ZR)])
        return carry

    lax.fori_loop(0, RPT // ZR, zcopy, 0)
    idx_in.wait()
    idx_in2.wait()
    plsc.subcore_barrier()

    my_x = x2_hbm.at[c]

    def start_g(j, rbuf, sem):
        pltpu.async_copy(my_x.at[sidx_all.at[j]], rbuf, sem)

    def wait_g(j, rbuf, sem):
        pltpu.make_async_copy(my_x.at[sidx_all.at[j]], rbuf, sem).wait()

    def scat(j, rbuf):
        pltpu.sync_copy(rbuf, acc.at[didx_all.at[j]], add=True)

        @pl.when(c == 0)
        def _():
            pltpu.async_copy(ones16, cnt.at[didx_all.at[j]], csem, add=True)

    start_g(0, rows0, gsem0)
    start_g(1, rows1, gsem1)

    def edge(k, carry):
        j0 = 2 * k
        wait_g(j0, rows0, gsem0)
        scat(j0, rows0)
        start_g(j0 + 2, rows0, gsem0)
        j1 = j0 + 1
        wait_g(j1, rows1, gsem1)
        scat(j1, rows1)
        start_g(j1 + 2, rows1, gsem1)
        return carry

    lax.fori_loop(0, NCHUNK // 2 - 1, edge, 0)
    wait_g(NCHUNK - 2, rows0, gsem0)
    scat(NCHUNK - 2, rows0)
    wait_g(NCHUNK - 1, rows1, gsem1)
    scat(NCHUNK - 1, rows1)

    @pl.when(c == 0)
    def _():
        def drain(j, carry):
            pltpu.make_async_copy(ones16, cnt.at[didx_all.at[j]], csem).wait()
            return carry

        lax.fori_loop(0, NCHUNK, drain, 0)

    plsc.subcore_barrier()

    def out_copy(k, carry):
        r0 = rbase + k * ZR
        pltpu.sync_copy(acc.at[pl.ds(r0, ZR)], zrows)
        pltpu.sync_copy(zrows, out_s.at[c, pl.ds(r0, ZR)])

        @pl.when(c == 0)
        def _():
            pltpu.sync_copy(cnt.at[pl.ds(r0, ZR)], z16)
            pltpu.sync_copy(z16, out_c.at[pl.ds(r0, ZR)])

        return carry

    lax.fori_loop(0, RPT // ZR, out_copy, 0)


_sc_segment_sum = pl.kernel(
    _sc_body,
    out_type=(
        jax.ShapeDtypeStruct((NC, NP, DH), jnp.float32),
        jax.ShapeDtypeStruct((NP, 16), jnp.float32),
    ),
    mesh=plsc.VectorSubcoreMesh(
        core_axis_name="c", subcore_axis_name="s",
        num_cores=NC, num_subcores=NS),
    compiler_params=pltpu.CompilerParams(use_tc_tiling_on_sc=False),
    scratch_types=[
        pltpu.VMEM((NCHUNK, CHUNK), jnp.int32),
        pltpu.VMEM((NCHUNK, CHUNK), jnp.int32),
        pltpu.VMEM((CHUNK, DH), jnp.float32),
        pltpu.VMEM((CHUNK, DH), jnp.float32),
        pltpu.VMEM((CHUNK, 16), jnp.float32),
        pltpu.VMEM((ZR, DH), jnp.float32),
        pltpu.VMEM((ZR, 16), jnp.float32),
        pltpu.SemaphoreType.DMA,
        pltpu.SemaphoreType.DMA,
        pltpu.SemaphoreType.DMA,
        pltpu.SemaphoreType.DMA,
        pltpu.VMEM_SHARED((NP, DH), jnp.float32),
        pltpu.VMEM_SHARED((NP, 16), jnp.float32),
    ],
)


R = 1000


def _tc_body(s_ref, c_ref, x_ref, wl_ref, wr_ref, b_ref, o_ref):
    ssum = jnp.concatenate([s_ref[0], s_ref[1]], axis=-1)
    cnt = c_ref[...][:, :1]
    mean = ssum / jnp.clip(cnt, 1.0, None)
    dn = (((1,), (1,)), ((), ()))
    out = (lax.dot_general(mean, wl_ref[...], dn,
                           preferred_element_type=jnp.float32)
           + lax.dot_general(x_ref[...], wr_ref[...], dn,
                             preferred_element_type=jnp.float32)
           + b_ref[...])
    nrm = jnp.sqrt(jnp.sum(out * out, axis=-1, keepdims=True))
    o_ref[...] = out / jnp.maximum(nrm, 1e-12)


def _tc_combine(ps, pc, x, W_l, W_r, b2):
    return pl.pallas_call(
        _tc_body,
        grid=(N // R,),
        in_specs=[
            pl.BlockSpec((NC, R, DH), lambda i: (0, i, 0)),
            pl.BlockSpec((R, 16), lambda i: (i, 0)),
            pl.BlockSpec((R, D), lambda i: (i, 0)),
            pl.BlockSpec((D, D), lambda i: (0, 0)),
            pl.BlockSpec((D, D), lambda i: (0, 0)),
            pl.BlockSpec((1, D), lambda i: (0, 0)),
        ],
        out_specs=pl.BlockSpec((R, D), lambda i: (i, 0)),
        out_shape=jax.ShapeDtypeStruct((N, D), jnp.float32),
    )(ps, pc, x, W_l, W_r, b2)


@jax.jit
def kernel(x, edge_index, W_l, W_r, b_l):
    src2 = edge_index[0].reshape(E // CHUNK, CHUNK)
    dst2 = edge_index[1].reshape(E // CHUNK, CHUNK)
    x2 = jnp.stack([x[:, :DH], x[:, DH:]])
    ps, pc = _sc_segment_sum(x2, src2, dst2)
    return _tc_combine(ps, pc, x, W_l, W_r, b_l.reshape(1, D))


# speedup vs baseline: 10.9053x; 1.0470x over previous
import jax
import jax.numpy as jnp
from jax import lax
from jax.experimental import pallas as pl
from jax.experimental.pallas import tpu as pltpu
from jax.experimental.pallas import tpu_sc as plsc

N = 10000
E = 320000
D = 128
DH = D // 2

NC = 2
NS = 16
CPT = E // NS
CHUNK = 80
NCHUNK = CPT // CHUNK
HCHUNK = NCHUNK // 2
NP = 10240
RPT = NP // NS
ZR = 128


def _sc_body(xv_hbm, src_hbm, dst_hbm, out_s, out_c,
             sidx_all, didx_all, rows0, rows1, ones16, zrows, z16,
             isem, gsem0, gsem1, csem, acc, cnt):
    c = lax.axis_index("c")
    s = lax.axis_index("s")
    jbase = s * NCHUNK
    rbase = s * RPT

    zero16 = jnp.zeros((16,), jnp.float32)
    one16 = jnp.ones((16,), jnp.float32)

    idx_in = pltpu.async_copy(src_hbm.at[pl.ds(jbase, NCHUNK)], sidx_all, isem)
    idx_in2 = pltpu.async_copy(dst_hbm.at[pl.ds(jbase, NCHUNK)], didx_all, isem)

    def fill_z(i, carry):
        for k in range(DH // 16):
            zrows[i, pl.ds(k * 16, 16)] = zero16
        z16[i, :] = zero16
        return carry

    lax.fori_loop(0, ZR, fill_z, 0)

    def fill_one(i, carry):
        ones16[i, :] = one16
        return carry

    lax.fori_loop(0, CHUNK, fill_one, 0)

    def zcopy(k, carry):
        r0 = rbase + k * ZR
        pltpu.sync_copy(zrows, acc.at[pl.ds(r0, ZR)])
        pltpu.sync_copy(z16, cnt.at[pl.ds(r0, ZR)])
        return carry

    lax.fori_loop(0, RPT // ZR, zcopy, 0)
    idx_in.wait()
    idx_in2.wait()

    cvec = jnp.full((16,), c, jnp.int32)

    def xform(r, carry):
        for g in range(CHUNK // 16):
            v = sidx_all[r, pl.ds(g * 16, 16)]
            sidx_all[r, pl.ds(g * 16, 16)] = v + v + cvec
        return carry

    lax.fori_loop(0, NCHUNK, xform, 0)
    plsc.subcore_barrier()

    def start_g(j, rbuf, sem):
        pltpu.async_copy(xv_hbm.at[sidx_all.at[j]], rbuf, sem)

    def wait_g(j, rbuf, sem):
        pltpu.make_async_copy(xv_hbm.at[sidx_all.at[j]], rbuf, sem).wait()

    def scat(j, rbuf):
        pltpu.sync_copy(rbuf, acc.at[didx_all.at[j]], add=True)

        @pl.when((j < HCHUNK) == (c == 0))
        def _():
            pltpu.async_copy(ones16, cnt.at[didx_all.at[j]], csem, add=True)

    start_g(0, rows0, gsem0)
    start_g(1, rows1, gsem1)

    def edge(k, carry):
        j0 = 2 * k
        wait_g(j0, rows0, gsem0)
        scat(j0, rows0)
        start_g(j0 + 2, rows0, gsem0)
        j1 = j0 + 1
        wait_g(j1, rows1, gsem1)
        scat(j1, rows1)
        start_g(j1 + 2, rows1, gsem1)
        return carry

    lax.fori_loop(0, NCHUNK // 2 - 1, edge, 0)
    wait_g(NCHUNK - 2, rows0, gsem0)
    scat(NCHUNK - 2, rows0)
    wait_g(NCHUNK - 1, rows1, gsem1)
    scat(NCHUNK - 1, rows1)

    def drain(k, carry):
        j = k + c * HCHUNK
        pltpu.make_async_copy(ones16, cnt.at[didx_all.at[j]], csem).wait()
        return carry

    lax.fori_loop(0, HCHUNK, drain, 0)

    plsc.subcore_barrier()

    def out_copy(k, carry):
        r0 = rbase + k * ZR
        pltpu.sync_copy(acc.at[pl.ds(r0, ZR)], zrows)
        pltpu.sync_copy(zrows, out_s.at[c, pl.ds(r0, ZR)])
        pltpu.sync_copy(cnt.at[pl.ds(r0, ZR)], z16)
        pltpu.sync_copy(z16, out_c.at[c, pl.ds(r0, ZR)])
        return carry

    lax.fori_loop(0, RPT // ZR, out_copy, 0)


_sc_segment_sum = pl.kernel(
    _sc_body,
    out_type=(
        jax.ShapeDtypeStruct((NC, NP, DH), jnp.float32),
        jax.ShapeDtypeStruct((NC, NP, 16), jnp.float32),
    ),
    mesh=plsc.VectorSubcoreMesh(
        core_axis_name="c", subcore_axis_name="s",
        num_cores=NC, num_subcores=NS),
    compiler_params=pltpu.CompilerParams(use_tc_tiling_on_sc=False),
    scratch_types=[
        pltpu.VMEM((NCHUNK, CHUNK), jnp.int32),
        pltpu.VMEM((NCHUNK, CHUNK), jnp.int32),
        pltpu.VMEM((CHUNK, DH), jnp.float32),
        pltpu.VMEM((CHUNK, DH), jnp.float32),
        pltpu.VMEM((CHUNK, 16), jnp.float32),
        pltpu.VMEM((ZR, DH), jnp.float32),
        pltpu.VMEM((ZR, 16), jnp.float32),
        pltpu.SemaphoreType.DMA,
        pltpu.SemaphoreType.DMA,
        pltpu.SemaphoreType.DMA,
        pltpu.SemaphoreType.DMA,
        pltpu.VMEM_SHARED((NP, DH), jnp.float32),
        pltpu.VMEM_SHARED((NP, 16), jnp.float32),
    ],
)


R = 1000


def _tc_xr_body(x_ref, wr_ref, b_ref, o_ref):
    dn = (((1,), (1,)), ((), ()))
    o_ref[...] = lax.dot_general(
        x_ref[...], wr_ref[...], dn,
        preferred_element_type=jnp.float32) + b_ref[...]


def _tc_xr(x, W_r, b2):
    return pl.pallas_call(
        _tc_xr_body,
        grid=(N // R,),
        in_specs=[
            pl.BlockSpec((R, D), lambda i: (i, 0)),
            pl.BlockSpec((D, D), lambda i: (0, 0)),
            pl.BlockSpec((1, D), lambda i: (0, 0)),
        ],
        out_specs=pl.BlockSpec((R, D), lambda i: (i, 0)),
        out_shape=jax.ShapeDtypeStruct((N, D), jnp.float32),
    )(x, W_r, b2)


def _tc_body(s_ref, c_ref, xr_ref, wl_ref, o_ref):
    ssum = jnp.concatenate([s_ref[0], s_ref[1]], axis=-1)
    cnt = (c_ref[0] + c_ref[1])[:, :1]
    mean = ssum / jnp.clip(cnt, 1.0, None)
    dn = (((1,), (1,)), ((), ()))
    out = lax.dot_general(mean, wl_ref[...], dn,
                          preferred_element_type=jnp.float32) + xr_ref[...]
    nrm = jnp.sqrt(jnp.sum(out * out, axis=-1, keepdims=True))
    o_ref[...] = out / jnp.maximum(nrm, 1e-12)


def _tc_combine(ps, pc, xr, W_l):
    return pl.pallas_call(
        _tc_body,
        grid=(N // R,),
        in_specs=[
            pl.BlockSpec((NC, R, DH), lambda i: (0, i, 0)),
            pl.BlockSpec((NC, R, 16), lambda i: (0, i, 0)),
            pl.BlockSpec((R, D), lambda i: (i, 0)),
            pl.BlockSpec((D, D), lambda i: (0, 0)),
        ],
        out_specs=pl.BlockSpec((R, D), lambda i: (i, 0)),
        out_shape=jax.ShapeDtypeStruct((N, D), jnp.float32),
    )(ps, pc, xr, W_l)


@jax.jit
def kernel(x, edge_index, W_l, W_r, b_l):
    src2 = edge_index[0].reshape(E // CHUNK, CHUNK)
    dst2 = edge_index[1].reshape(E // CHUNK, CHUNK)
    xv = x.reshape(2 * N, DH)
    xr = _tc_xr(x, W_r, b_l.reshape(1, D))
    ps, pc = _sc_segment_sum(xv, src2, dst2)
    return _tc_combine(ps, pc, xr, W_l)


# speedup vs baseline: 13.3561x; 1.2247x over previous
import jax
import jax.numpy as jnp
from jax import lax
from jax.experimental import pallas as pl
from jax.experimental.pallas import tpu as pltpu
from jax.experimental.pallas import tpu_sc as plsc

N = 10000
E = 320000
D = 128
DH = D // 2

NC = 2
NS = 16
CPT = E // NS
CHUNK = 80
NCHUNK = CPT // CHUNK
HCHUNK = NCHUNK // 2
NP = 10240
RPT = NP // NS
ZR = 128


NB = 5


def _sc_body(xv_hbm, src_hbm, dst_hbm, out_s, out_c,
             sidx_all, didx_all, rows, ones16, zrows, z16,
             isem, gsem, ssem, csem, zsem, acc, cnt):
    c = lax.axis_index("c")
    s = lax.axis_index("s")
    jbase = s * NCHUNK
    rbase = s * RPT

    zero16 = jnp.zeros((16,), jnp.float32)
    one16 = jnp.ones((16,), jnp.float32)

    idx_in = pltpu.async_copy(src_hbm.at[pl.ds(jbase, NCHUNK)], sidx_all, isem)
    idx_in2 = pltpu.async_copy(dst_hbm.at[pl.ds(jbase, NCHUNK)], didx_all, isem)

    def fill_z(i, carry):
        for k in range(DH // 16):
            zrows[0, i, pl.ds(k * 16, 16)] = zero16
        z16[0, i, :] = zero16
        return carry

    lax.fori_loop(0, ZR, fill_z, 0)

    def fill_one(i, carry):
        ones16[i, :] = one16
        return carry

    lax.fori_loop(0, CHUNK, fill_one, 0)

    def zcopy(k, carry):
        r0 = rbase + k * ZR
        pltpu.async_copy(zrows.at[0], acc.at[pl.ds(r0, ZR)], zsem)
        pltpu.async_copy(z16.at[0], cnt.at[pl.ds(r0, ZR)], zsem)
        return carry

    lax.fori_loop(0, RPT // ZR, zcopy, 0)
    idx_in.wait()
    idx_in2.wait()

    cvec = jnp.full((16,), c, jnp.int32)

    def xform(r, carry):
        for g in range(CHUNK // 16):
            v = sidx_all[r, pl.ds(g * 16, 16)]
            sidx_all[r, pl.ds(g * 16, 16)] = v + v + cvec
        return carry

    lax.fori_loop(0, NCHUNK, xform, 0)

    def zdrain(k, carry):
        r0 = rbase + k * ZR
        pltpu.make_async_copy(zrows.at[0], acc.at[pl.ds(r0, ZR)], zsem).wait()
        pltpu.make_async_copy(z16.at[0], cnt.at[pl.ds(r0, ZR)], zsem).wait()
        return carry

    lax.fori_loop(0, RPT // ZR, zdrain, 0)
    plsc.subcore_barrier()

    def start_g(j, b):
        pltpu.async_copy(xv_hbm.at[sidx_all.at[j]], rows.at[b], gsem.at[b])

    def wait_g(j, b):
        pltpu.make_async_copy(
            xv_hbm.at[sidx_all.at[j]], rows.at[b], gsem.at[b]).wait()

    def start_s(j, b):
        pltpu.async_copy(rows.at[b], acc.at[didx_all.at[j]], ssem.at[b],
                         add=True)

        @pl.when((j < HCHUNK) == (c == 0))
        def _():
            pltpu.async_copy(ones16, cnt.at[didx_all.at[j]], csem, add=True)

    def wait_s(j, b):
        pltpu.make_async_copy(
            rows.at[b], acc.at[didx_all.at[j]], ssem.at[b]).wait()

    for b in range(NB):
        start_g(b, b)

    def edge(k, carry):
        j = NB * k
        for b in range(NB):
            wait_g(j + b, b)
            start_s(j + b, b)
        for b in range(NB):
            wait_s(j + b, b)
            start_g(j + NB + b, b)
        return carry

    lax.fori_loop(0, NCHUNK // NB - 1, edge, 0)
    jlast = NCHUNK - NB
    for b in range(NB):
        wait_g(jlast + b, b)
        start_s(jlast + b, b)
    for b in range(NB):
        wait_s(jlast + b, b)

    def drain(k, carry):
        j = k + c * HCHUNK
        pltpu.make_async_copy(ones16, cnt.at[didx_all.at[j]], csem).wait()
        return carry

    lax.fori_loop(0, HCHUNK, drain, 0)

    plsc.subcore_barrier()

    for k in range(RPT // ZR):
        r0 = rbase + k * ZR
        pltpu.sync_copy(acc.at[pl.ds(r0, ZR)], zrows.at[0])
        pltpu.sync_copy(zrows.at[0], out_s.at[c, pl.ds(r0, ZR)])
        pltpu.sync_copy(cnt.at[pl.ds(r0, ZR)], z16.at[0])
        pltpu.sync_copy(z16.at[0], out_c.at[c, pl.ds(r0, ZR)])


_sc_segment_sum = pl.kernel(
    _sc_body,
    out_type=(
        jax.ShapeDtypeStruct((NC, NP, DH), jnp.float32),
        jax.ShapeDtypeStruct((NC, NP, 16), jnp.float32),
    ),
    mesh=plsc.VectorSubcoreMesh(
        core_axis_name="c", subcore_axis_name="s",
        num_cores=NC, num_subcores=NS),
    compiler_params=pltpu.CompilerParams(use_tc_tiling_on_sc=False),
    scratch_types=[
        pltpu.VMEM((NCHUNK, CHUNK), jnp.int32),
        pltpu.VMEM((NCHUNK, CHUNK), jnp.int32),
        pltpu.VMEM((NB, CHUNK, DH), jnp.float32),
        pltpu.VMEM((CHUNK, 16), jnp.float32),
        pltpu.VMEM((1, ZR, DH), jnp.float32),
        pltpu.VMEM((1, ZR, 16), jnp.float32),
        pltpu.SemaphoreType.DMA,
        pltpu.SemaphoreType.DMA((NB,)),
        pltpu.SemaphoreType.DMA((NB,)),
        pltpu.SemaphoreType.DMA,
        pltpu.SemaphoreType.DMA,
        pltpu.VMEM_SHARED((NP, DH), jnp.float32),
        pltpu.VMEM_SHARED((NP, 16), jnp.float32),
    ],
)


R = 1000


def _tc_xr_body(x_ref, wr_ref, b_ref, o_ref):
    dn = (((1,), (1,)), ((), ()))
    o_ref[...] = lax.dot_general(
        x_ref[...], wr_ref[...], dn,
        preferred_element_type=jnp.float32) + b_ref[...]


def _tc_xr(x, W_r, b2):
    return pl.pallas_call(
        _tc_xr_body,
        grid=(N // R,),
        in_specs=[
            pl.BlockSpec((R, D), lambda i: (i, 0)),
            pl.BlockSpec((D, D), lambda i: (0, 0)),
            pl.BlockSpec((1, D), lambda i: (0, 0)),
        ],
        out_specs=pl.BlockSpec((R, D), lambda i: (i, 0)),
        out_shape=jax.ShapeDtypeStruct((N, D), jnp.float32),
    )(x, W_r, b2)


def _tc_body(s_ref, c_ref, xr_ref, wl_ref, o_ref):
    ssum = jnp.concatenate([s_ref[0], s_ref[1]], axis=-1)
    cnt = (c_ref[0] + c_ref[1])[:, :1]
    mean = ssum / jnp.clip(cnt, 1.0, None)
    dn = (((1,), (1,)), ((), ()))
    out = lax.dot_general(mean, wl_ref[...], dn,
                          preferred_element_type=jnp.float32) + xr_ref[...]
    nrm = jnp.sqrt(jnp.sum(out * out, axis=-1, keepdims=True))
    o_ref[...] = out / jnp.maximum(nrm, 1e-12)


def _tc_combine(ps, pc, xr, W_l):
    return pl.pallas_call(
        _tc_body,
        grid=(N // R,),
        in_specs=[
            pl.BlockSpec((NC, R, DH), lambda i: (0, i, 0)),
            pl.BlockSpec((NC, R, 16), lambda i: (0, i, 0)),
            pl.BlockSpec((R, D), lambda i: (i, 0)),
            pl.BlockSpec((D, D), lambda i: (0, 0)),
        ],
        out_specs=pl.BlockSpec((R, D), lambda i: (i, 0)),
        out_shape=jax.ShapeDtypeStruct((N, D), jnp.float32),
    )(ps, pc, xr, W_l)


@jax.jit
def kernel(x, edge_index, W_l, W_r, b_l):
    src2 = edge_index[0].reshape(E // CHUNK, CHUNK)
    dst2 = edge_index[1].reshape(E // CHUNK, CHUNK)
    xv = x.reshape(2 * N, DH)
    xr = _tc_xr(x, W_r, b_l.reshape(1, D))
    ps, pc = _sc_segment_sum(xv, src2, dst2)
    return _tc_combine(ps, pc, xr, W_l)


# speedup vs baseline: 13.4302x; 1.0055x over previous
import jax
import jax.numpy as jnp
from jax import lax
from jax.experimental import pallas as pl
from jax.experimental.pallas import tpu as pltpu
from jax.experimental.pallas import tpu_sc as plsc

N = 10000
E = 320000
D = 128
DH = D // 2

NC = 2
NS = 16
CPT = E // NS
CHUNK = 80
NCHUNK = CPT // CHUNK
HCHUNK = NCHUNK // 2
NP = 10240
RPT = NP // NS
ZR = 128


NB = 5


def _sc_body(xv_hbm, src_hbm, dst_hbm, out_s, out_c,
             sidx_all, didx_all, rows, ones16, zrows, z16,
             isem, gsem, ssem, csem, zsem, acc, cnt):
    c = lax.axis_index("c")
    s = lax.axis_index("s")
    jbase = s * NCHUNK
    rbase = s * RPT

    zero16 = jnp.zeros((16,), jnp.float32)
    one16 = jnp.ones((16,), jnp.float32)

    idx_in = pltpu.async_copy(src_hbm.at[pl.ds(jbase, NCHUNK)], sidx_all, isem)
    idx_in2 = pltpu.async_copy(dst_hbm.at[pl.ds(jbase, NCHUNK)], didx_all, isem)

    def fill_z(i, carry):
        for k in range(DH // 16):
            zrows[0, i, pl.ds(k * 16, 16)] = zero16
        z16[0, i, :] = zero16
        return carry

    lax.fori_loop(0, ZR, fill_z, 0)

    def fill_one(i, carry):
        ones16[i, :] = one16
        return carry

    lax.fori_loop(0, CHUNK, fill_one, 0)

    def zcopy(k, carry):
        r0 = rbase + k * ZR
        pltpu.async_copy(zrows.at[0], acc.at[pl.ds(r0, ZR)], zsem)
        pltpu.async_copy(z16.at[0], cnt.at[pl.ds(r0, ZR)], zsem)
        return carry

    lax.fori_loop(0, RPT // ZR, zcopy, 0)
    idx_in.wait()
    idx_in2.wait()

    cvec = jnp.full((16,), c, jnp.int32)

    def xform(r, carry):
        for g in range(CHUNK // 16):
            v = sidx_all[r, pl.ds(g * 16, 16)]
            sidx_all[r, pl.ds(g * 16, 16)] = v + v + cvec
        return carry

    lax.fori_loop(0, NCHUNK, xform, 0)

    def zdrain(k, carry):
        r0 = rbase + k * ZR
        pltpu.make_async_copy(zrows.at[0], acc.at[pl.ds(r0, ZR)], zsem).wait()
        pltpu.make_async_copy(z16.at[0], cnt.at[pl.ds(r0, ZR)], zsem).wait()
        return carry

    lax.fori_loop(0, RPT // ZR, zdrain, 0)
    plsc.subcore_barrier()

    def start_g(j, b):
        pltpu.async_copy(xv_hbm.at[sidx_all.at[j]], rows.at[b], gsem.at[b])

    def wait_g(j, b):
        pltpu.make_async_copy(
            xv_hbm.at[sidx_all.at[j]], rows.at[b], gsem.at[b]).wait()

    def start_s(j, b):
        pltpu.async_copy(rows.at[b], acc.at[didx_all.at[j]], ssem.at[b],
                         add=True)

        @pl.when((j < HCHUNK) == (c == 0))
        def _():
            pltpu.async_copy(ones16, cnt.at[didx_all.at[j]], csem, add=True)

    def wait_s(j, b):
        pltpu.make_async_copy(
            rows.at[b], acc.at[didx_all.at[j]], ssem.at[b]).wait()

    for b in range(NB):
        start_g(b, b)

    def edge(k, carry):
        j = NB * k
        for b in range(NB):
            wait_g(j + b, b)
            start_s(j + b, b)
        for b in range(NB):
            wait_s(j + b, b)
            start_g(j + NB + b, b)
        return carry

    lax.fori_loop(0, NCHUNK // NB - 1, edge, 0)
    jlast = NCHUNK - NB
    for b in range(NB):
        wait_g(jlast + b, b)
        start_s(jlast + b, b)
    for b in range(NB):
        wait_s(jlast + b, b)

    def drain(k, carry):
        j = k + c * HCHUNK
        pltpu.make_async_copy(ones16, cnt.at[didx_all.at[j]], csem).wait()
        return carry

    lax.fori_loop(0, HCHUNK, drain, 0)

    plsc.subcore_barrier()

    for k in range(RPT // ZR):
        r0 = rbase + k * ZR
        pltpu.sync_copy(acc.at[pl.ds(r0, ZR)], zrows.at[0])
        pltpu.sync_copy(zrows.at[0], out_s.at[c, pl.ds(r0, ZR)])
        pltpu.sync_copy(cnt.at[pl.ds(r0, ZR)], z16.at[0])
        pltpu.sync_copy(z16.at[0], out_c.at[c, pl.ds(r0, ZR)])


_sc_segment_sum = pl.kernel(
    _sc_body,
    out_type=(
        jax.ShapeDtypeStruct((NC, NP, DH), jnp.float32),
        jax.ShapeDtypeStruct((NC, NP, 16), jnp.float32),
    ),
    mesh=plsc.VectorSubcoreMesh(
        core_axis_name="c", subcore_axis_name="s",
        num_cores=NC, num_subcores=NS),
    compiler_params=pltpu.CompilerParams(use_tc_tiling_on_sc=False),
    scratch_types=[
        pltpu.VMEM((NCHUNK, CHUNK), jnp.int32),
        pltpu.VMEM((NCHUNK, CHUNK), jnp.int32),
        pltpu.VMEM((NB, CHUNK, DH), jnp.float32),
        pltpu.VMEM((CHUNK, 16), jnp.float32),
        pltpu.VMEM((1, ZR, DH), jnp.float32),
        pltpu.VMEM((1, ZR, 16), jnp.float32),
        pltpu.SemaphoreType.DMA,
        pltpu.SemaphoreType.DMA((NB,)),
        pltpu.SemaphoreType.DMA((NB,)),
        pltpu.SemaphoreType.DMA,
        pltpu.SemaphoreType.DMA,
        pltpu.VMEM_SHARED((NP, DH), jnp.float32),
        pltpu.VMEM_SHARED((NP, 16), jnp.float32),
    ],
)


R = 1000


def _tc_xr_body(x_ref, wr_ref, b_ref, o_ref):
    dn = (((1,), (1,)), ((), ()))
    o_ref[...] = lax.dot_general(
        x_ref[...], wr_ref[...], dn,
        preferred_element_type=jnp.float32) + b_ref[...]


def _tc_xr(x, W_r, b2):
    return pl.pallas_call(
        _tc_xr_body,
        grid=(N // R,),
        in_specs=[
            pl.BlockSpec((R, D), lambda i: (i, 0)),
            pl.BlockSpec((D, D), lambda i: (0, 0)),
            pl.BlockSpec((1, D), lambda i: (0, 0)),
        ],
        out_specs=pl.BlockSpec((R, D), lambda i: (i, 0)),
        out_shape=jax.ShapeDtypeStruct((N, D), jnp.float32),
    )(x, W_r, b2)


def _tc_body(s_ref, c_ref, x_ref, wl_ref, wr_ref, b_ref, o_ref):
    ssum = jnp.concatenate([s_ref[0], s_ref[1]], axis=-1)
    cnt = (c_ref[0] + c_ref[1])[:, :1]
    mean = ssum / jnp.clip(cnt, 1.0, None)
    dn = (((1,), (1,)), ((), ()))
    out = (lax.dot_general(mean, wl_ref[...], dn,
                           preferred_element_type=jnp.float32)
           + lax.dot_general(x_ref[...], wr_ref[...], dn,
                             preferred_element_type=jnp.float32)
           + b_ref[...])
    nrm = jnp.sqrt(jnp.sum(out * out, axis=-1, keepdims=True))
    o_ref[...] = out / jnp.maximum(nrm, 1e-12)


def _tc_combine(ps, pc, x, W_l, W_r, b2):
    return pl.pallas_call(
        _tc_body,
        grid=(N // R,),
        in_specs=[
            pl.BlockSpec((NC, R, DH), lambda i: (0, i, 0)),
            pl.BlockSpec((NC, R, 16), lambda i: (0, i, 0)),
            pl.BlockSpec((R, D), lambda i: (i, 0)),
            pl.BlockSpec((D, D), lambda i: (0, 0)),
            pl.BlockSpec((D, D), lambda i: (0, 0)),
            pl.BlockSpec((1, D), lambda i: (0, 0)),
        ],
        out_specs=pl.BlockSpec((R, D), lambda i: (i, 0)),
        out_shape=jax.ShapeDtypeStruct((N, D), jnp.float32),
    )(ps, pc, x, W_l, W_r, b2)


@jax.jit
def kernel(x, edge_index, W_l, W_r, b_l):
    src2 = edge_index[0].reshape(E // CHUNK, CHUNK)
    dst2 = edge_index[1].reshape(E // CHUNK, CHUNK)
    xv = x.reshape(2 * N, DH)
    ps, pc = _sc_segment_sum(xv, src2, dst2)
    return _tc_combine(ps, pc, x, W_l, W_r, b_l.reshape(1, D))


# speedup vs baseline: 14.1638x; 1.0546x over previous
import jax
import jax.numpy as jnp
from jax import lax
from jax.experimental import pallas as pl
from jax.experimental.pallas import tpu as pltpu
from jax.experimental.pallas import tpu_sc as plsc

N = 10000
E = 320000
D = 128
DH = D // 2

NC = 2
NS = 16
CPT = E // NS
CHUNK = 80
NCHUNK = CPT // CHUNK
HCHUNK = NCHUNK // 2
NP = 10240
RPT = NP // NS
ZR = 128


NB = 5


def _sc_body(xv_hbm, src_hbm, dst_hbm, out_s, out_c,
             sidx_all, didx_all, rows, ones16, zrows, z16,
             isem, gsem, ssem, csem, zsem, acc, cnt):
    c = lax.axis_index("c")
    s = lax.axis_index("s")
    jbase = s * NCHUNK
    rbase = s * RPT

    zero16 = jnp.zeros((16,), jnp.float32)
    one16 = jnp.ones((16,), jnp.float32)

    idx_in = pltpu.async_copy(src_hbm.at[pl.ds(jbase, NCHUNK)], sidx_all, isem)
    idx_in2 = pltpu.async_copy(dst_hbm.at[pl.ds(jbase, NCHUNK)], didx_all, isem)

    def fill_z(i, carry):
        for k in range(DH // 16):
            zrows[0, i, pl.ds(k * 16, 16)] = zero16
        z16[0, i, :] = zero16
        return carry

    lax.fori_loop(0, ZR, fill_z, 0)

    def fill_one(i, carry):
        ones16[i, :] = one16
        return carry

    lax.fori_loop(0, CHUNK, fill_one, 0)

    def zcopy(k, carry):
        r0 = rbase + k * ZR
        pltpu.async_copy(zrows.at[0], acc.at[pl.ds(r0, ZR)], zsem)
        pltpu.async_copy(z16.at[0], cnt.at[pl.ds(r0, ZR)], zsem)
        return carry

    lax.fori_loop(0, RPT // ZR, zcopy, 0)
    idx_in.wait()
    idx_in2.wait()

    cvec = jnp.full((16,), c, jnp.int32)

    def xform(r, carry):
        for g in range(CHUNK // 16):
            v = sidx_all[r, pl.ds(g * 16, 16)]
            sidx_all[r, pl.ds(g * 16, 16)] = v + v + cvec
        return carry

    lax.fori_loop(0, NCHUNK, xform, 0)

    def zdrain(k, carry):
        r0 = rbase + k * ZR
        pltpu.make_async_copy(zrows.at[0], acc.at[pl.ds(r0, ZR)], zsem).wait()
        pltpu.make_async_copy(z16.at[0], cnt.at[pl.ds(r0, ZR)], zsem).wait()
        return carry

    lax.fori_loop(0, RPT // ZR, zdrain, 0)
    plsc.subcore_barrier()

    def start_g(j, b):
        pltpu.async_copy(xv_hbm.at[sidx_all.at[j]], rows.at[b], gsem.at[b])

    def wait_g(j, b):
        pltpu.make_async_copy(
            xv_hbm.at[sidx_all.at[j]], rows.at[b], gsem.at[b]).wait()

    def start_s(j, b):
        pltpu.async_copy(rows.at[b], acc.at[didx_all.at[j]], ssem.at[b],
                         add=True)

        @pl.when((j < HCHUNK) == (c == 0))
        def _():
            pltpu.async_copy(ones16, cnt.at[didx_all.at[j]], csem, add=True)

    def wait_s(j, b):
        pltpu.make_async_copy(
            rows.at[b], acc.at[didx_all.at[j]], ssem.at[b]).wait()

    for b in range(NB):
        start_g(b, b)

    def edge(k, carry):
        j = NB * k
        for b in range(NB):
            wait_g(j + b, b)
            start_s(j + b, b)
        for b in range(NB):
            wait_s(j + b, b)
            start_g(j + NB + b, b)
        return carry

    lax.fori_loop(0, NCHUNK // NB - 1, edge, 0)
    jlast = NCHUNK - NB
    for b in range(NB):
        wait_g(jlast + b, b)
        start_s(jlast + b, b)
    for b in range(NB):
        wait_s(jlast + b, b)

    def drain(k, carry):
        j = k + c * HCHUNK
        pltpu.make_async_copy(ones16, cnt.at[didx_all.at[j]], csem).wait()
        return carry

    lax.fori_loop(0, HCHUNK, drain, 0)

    plsc.subcore_barrier()

    for k in range(RPT // ZR):
        r0 = rbase + k * ZR
        pltpu.sync_copy(acc.at[pl.ds(r0, ZR)], zrows.at[0])
        pltpu.sync_copy(zrows.at[0],
                        out_s.at[pl.ds(r0, ZR), pl.ds(c * DH, DH)])
        pltpu.sync_copy(cnt.at[pl.ds(r0, ZR)], z16.at[0])
        pltpu.sync_copy(z16.at[0], out_c.at[c, pl.ds(r0, ZR)])


_sc_segment_sum = pl.kernel(
    _sc_body,
    out_type=(
        jax.ShapeDtypeStruct((NP, D), jnp.float32),
        jax.ShapeDtypeStruct((NC, NP, 16), jnp.float32),
    ),
    mesh=plsc.VectorSubcoreMesh(
        core_axis_name="c", subcore_axis_name="s",
        num_cores=NC, num_subcores=NS),
    compiler_params=pltpu.CompilerParams(use_tc_tiling_on_sc=False),
    scratch_types=[
        pltpu.VMEM((NCHUNK, CHUNK), jnp.int32),
        pltpu.VMEM((NCHUNK, CHUNK), jnp.int32),
        pltpu.VMEM((NB, CHUNK, DH), jnp.float32),
        pltpu.VMEM((CHUNK, 16), jnp.float32),
        pltpu.VMEM((1, ZR, DH), jnp.float32),
        pltpu.VMEM((1, ZR, 16), jnp.float32),
        pltpu.SemaphoreType.DMA,
        pltpu.SemaphoreType.DMA((NB,)),
        pltpu.SemaphoreType.DMA((NB,)),
        pltpu.SemaphoreType.DMA,
        pltpu.SemaphoreType.DMA,
        pltpu.VMEM_SHARED((NP, DH), jnp.float32),
        pltpu.VMEM_SHARED((NP, 16), jnp.float32),
    ],
)


R = 1000


def _tc_xr_body(x_ref, wr_ref, b_ref, o_ref):
    dn = (((1,), (1,)), ((), ()))
    o_ref[...] = lax.dot_general(
        x_ref[...], wr_ref[...], dn,
        preferred_element_type=jnp.float32) + b_ref[...]


def _tc_xr(x, W_r, b2):
    return pl.pallas_call(
        _tc_xr_body,
        grid=(N // R,),
        in_specs=[
            pl.BlockSpec((R, D), lambda i: (i, 0)),
            pl.BlockSpec((D, D), lambda i: (0, 0)),
            pl.BlockSpec((1, D), lambda i: (0, 0)),
        ],
        out_specs=pl.BlockSpec((R, D), lambda i: (i, 0)),
        out_shape=jax.ShapeDtypeStruct((N, D), jnp.float32),
    )(x, W_r, b2)


def _tc_body(s_ref, c_ref, x_ref, wl_ref, wr_ref, b_ref, o_ref):
    ssum = s_ref[...]
    cnt = (c_ref[0] + c_ref[1])[:, :1]
    mean = ssum / jnp.clip(cnt, 1.0, None)
    dn = (((1,), (1,)), ((), ()))
    out = (lax.dot_general(mean, wl_ref[...], dn,
                           preferred_element_type=jnp.float32)
           + lax.dot_general(x_ref[...], wr_ref[...], dn,
                             preferred_element_type=jnp.float32)
           + b_ref[...])
    nrm = jnp.sqrt(jnp.sum(out * out, axis=-1, keepdims=True))
    o_ref[...] = out / jnp.maximum(nrm, 1e-12)


def _tc_combine(ps, pc, x, W_l, W_r, b2):
    return pl.pallas_call(
        _tc_body,
        grid=(N // R,),
        in_specs=[
            pl.BlockSpec((R, D), lambda i: (i, 0)),
            pl.BlockSpec((NC, R, 16), lambda i: (0, i, 0)),
            pl.BlockSpec((R, D), lambda i: (i, 0)),
            pl.BlockSpec((D, D), lambda i: (0, 0)),
            pl.BlockSpec((D, D), lambda i: (0, 0)),
            pl.BlockSpec((1, D), lambda i: (0, 0)),
        ],
        out_specs=pl.BlockSpec((R, D), lambda i: (i, 0)),
        out_shape=jax.ShapeDtypeStruct((N, D), jnp.float32),
    )(ps, pc, x, W_l, W_r, b2)


@jax.jit
def kernel(x, edge_index, W_l, W_r, b_l):
    src2 = edge_index[0].reshape(E // CHUNK, CHUNK)
    dst2 = edge_index[1].reshape(E // CHUNK, CHUNK)
    xv = x.reshape(2 * N, DH)
    ps, pc = _sc_segment_sum(xv, src2, dst2)
    return _tc_combine(ps, pc, x, W_l, W_r, b_l.reshape(1, D))


# speedup vs baseline: 16.1048x; 1.1370x over previous
import jax
import jax.numpy as jnp
from jax import lax
from jax.experimental import pallas as pl
from jax.experimental.pallas import tpu as pltpu
from jax.experimental.pallas import tpu_sc as plsc

N = 10000
E = 320000
D = 128
DH = D // 2

NC = 2
NS = 16
CPT = E // NS
CHUNK = 80
NCHUNK = CPT // CHUNK
HCHUNK = NCHUNK // 2
NP = 10240
RPT = NP // NS
ZR = 128


NB = 5


def _sc_body(xv_hbm, eidx_hbm, out_s, out_c,
             sidx_all, didx_all, rows, ones16, zrows, z16,
             isem, gsem, ssem, csem, zsem, acc, cnt):
    c = lax.axis_index("c")
    s = lax.axis_index("s")
    jbase = s * NCHUNK
    rbase = s * RPT

    zero16 = jnp.zeros((16,), jnp.float32)
    one16 = jnp.ones((16,), jnp.float32)

    idx_in = pltpu.async_copy(
        eidx_hbm.at[pl.ds(jbase, NCHUNK)], sidx_all, isem)
    idx_in2 = pltpu.async_copy(
        eidx_hbm.at[pl.ds(E // CHUNK + jbase, NCHUNK)], didx_all, isem)

    def fill_z(i, carry):
        for k in range(DH // 16):
            zrows[0, i, pl.ds(k * 16, 16)] = zero16
        z16[0, i, :] = zero16
        return carry

    lax.fori_loop(0, ZR, fill_z, 0)

    def fill_one(i, carry):
        ones16[i, :] = one16
        return carry

    lax.fori_loop(0, CHUNK, fill_one, 0)

    def zcopy(k, carry):
        r0 = rbase + k * ZR
        pltpu.async_copy(zrows.at[0], acc.at[pl.ds(r0, ZR)], zsem)
        pltpu.async_copy(z16.at[0], cnt.at[pl.ds(r0, ZR)], zsem)
        return carry

    lax.fori_loop(0, RPT // ZR, zcopy, 0)
    idx_in.wait()
    idx_in2.wait()

    cvec = jnp.full((16,), c, jnp.int32)

    def xform(r, carry):
        for g in range(CHUNK // 16):
            v = sidx_all[r, pl.ds(g * 16, 16)]
            sidx_all[r, pl.ds(g * 16, 16)] = v + v + cvec
        return carry

    lax.fori_loop(0, NCHUNK, xform, 0)

    def zdrain(k, carry):
        r0 = rbase + k * ZR
        pltpu.make_async_copy(zrows.at[0], acc.at[pl.ds(r0, ZR)], zsem).wait()
        pltpu.make_async_copy(z16.at[0], cnt.at[pl.ds(r0, ZR)], zsem).wait()
        return carry

    lax.fori_loop(0, RPT // ZR, zdrain, 0)
    plsc.subcore_barrier()

    def start_g(j, b):
        pltpu.async_copy(xv_hbm.at[sidx_all.at[j]], rows.at[b], gsem.at[b])

    def wait_g(j, b):
        pltpu.make_async_copy(
            xv_hbm.at[sidx_all.at[j]], rows.at[b], gsem.at[b]).wait()

    def start_s(j, b):
        pltpu.async_copy(rows.at[b], acc.at[didx_all.at[j]], ssem.at[b],
                         add=True)

        @pl.when((j < HCHUNK) == (c == 0))
        def _():
            pltpu.async_copy(ones16, cnt.at[didx_all.at[j]], csem, add=True)

    def wait_s(j, b):
        pltpu.make_async_copy(
            rows.at[b], acc.at[didx_all.at[j]], ssem.at[b]).wait()

    for b in range(NB):
        start_g(b, b)

    def edge(k, carry):
        j = NB * k
        for b in range(NB):
            wait_g(j + b, b)
            start_s(j + b, b)
        for b in range(NB):
            wait_s(j + b, b)
            start_g(j + NB + b, b)
        return carry

    lax.fori_loop(0, NCHUNK // NB - 1, edge, 0)
    jlast = NCHUNK - NB
    for b in range(NB):
        wait_g(jlast + b, b)
        start_s(jlast + b, b)
    for b in range(NB):
        wait_s(jlast + b, b)

    def drain(k, carry):
        j = k + c * HCHUNK
        pltpu.make_async_copy(ones16, cnt.at[didx_all.at[j]], csem).wait()
        return carry

    lax.fori_loop(0, HCHUNK, drain, 0)

    plsc.subcore_barrier()

    for k in range(RPT // ZR):
        r0 = rbase + k * ZR
        pltpu.sync_copy(acc.at[pl.ds(r0, ZR)], zrows.at[0])
        pltpu.sync_copy(zrows.at[0],
                        out_s.at[pl.ds(r0, ZR), pl.ds(c * DH, DH)])
        pltpu.sync_copy(cnt.at[pl.ds(r0, ZR)], z16.at[0])
        pltpu.sync_copy(z16.at[0], out_c.at[c, pl.ds(r0, ZR)])


_sc_segment_sum = pl.kernel(
    _sc_body,
    out_type=(
        jax.ShapeDtypeStruct((NP, D), jnp.float32),
        jax.ShapeDtypeStruct((NC, NP, 16), jnp.float32),
    ),
    mesh=plsc.VectorSubcoreMesh(
        core_axis_name="c", subcore_axis_name="s",
        num_cores=NC, num_subcores=NS),
    compiler_params=pltpu.CompilerParams(use_tc_tiling_on_sc=False),
    scratch_types=[
        pltpu.VMEM((NCHUNK, CHUNK), jnp.int32),
        pltpu.VMEM((NCHUNK, CHUNK), jnp.int32),
        pltpu.VMEM((NB, CHUNK, DH), jnp.float32),
        pltpu.VMEM((CHUNK, 16), jnp.float32),
        pltpu.VMEM((1, ZR, DH), jnp.float32),
        pltpu.VMEM((1, ZR, 16), jnp.float32),
        pltpu.SemaphoreType.DMA,
        pltpu.SemaphoreType.DMA((NB,)),
        pltpu.SemaphoreType.DMA((NB,)),
        pltpu.SemaphoreType.DMA,
        pltpu.SemaphoreType.DMA,
        pltpu.VMEM_SHARED((NP, DH), jnp.float32),
        pltpu.VMEM_SHARED((NP, 16), jnp.float32),
    ],
)


R = 1024
RB = R // 8


def _tc_body(s_ref, c0_ref, c1_ref, x_ref, wl_ref, wr_ref, b_ref, o_ref):
    ssum = s_ref[...]
    c8 = c0_ref[...] + c1_ref[...]
    rep = jnp.repeat(c8, 8, axis=0)
    p_row = lax.broadcasted_iota(jnp.int32, (R, D), 0)
    q_col = lax.broadcasted_iota(jnp.int32, (R, D), 1)
    sel = (q_col // 16) == (p_row % 8)
    cnt = jnp.sum(jnp.where(sel, rep, 0.0), axis=-1,
                  keepdims=True) * (1.0 / 16.0)
    mean = ssum / jnp.clip(cnt, 1.0, None)
    dn = (((1,), (1,)), ((), ()))
    out = (lax.dot_general(mean, wl_ref[...], dn,
                           preferred_element_type=jnp.float32)
           + lax.dot_general(x_ref[...], wr_ref[...], dn,
                             preferred_element_type=jnp.float32)
           + b_ref[...])
    nrm = jnp.sqrt(jnp.sum(out * out, axis=-1, keepdims=True))
    o_ref[...] = out / jnp.maximum(nrm, 1e-12)


_CB = NP // 8 // RB


def _tc_combine(ps, pcv, x, W_l, W_r, b2):
    return pl.pallas_call(
        _tc_body,
        grid=(N // R + 1,),
        in_specs=[
            pl.BlockSpec((R, D), lambda i: (i, 0)),
            pl.BlockSpec((RB, D), lambda i: (i, 0)),
            pl.BlockSpec((RB, D), lambda i: (i + _CB, 0)),
            pl.BlockSpec((R, D), lambda i: (i, 0)),
            pl.BlockSpec((D, D), lambda i: (0, 0)),
            pl.BlockSpec((D, D), lambda i: (0, 0)),
            pl.BlockSpec((1, D), lambda i: (0, 0)),
        ],
        out_specs=pl.BlockSpec((R, D), lambda i: (i, 0)),
        out_shape=jax.ShapeDtypeStruct((N, D), jnp.float32),
    )(ps, pcv, pcv, x, W_l, W_r, b2)


@jax.jit
def kernel(x, edge_index, W_l, W_r, b_l):
    e2 = edge_index.reshape(2 * (E // CHUNK), CHUNK)
    xv = x.reshape(2 * N, DH)
    ps, pc = _sc_segment_sum(xv, e2)
    pcv = pc.reshape(NC * NP // 8, D)
    return _tc_combine(ps, pcv, x, W_l, W_r, b_l.reshape(1, D))


# speedup vs baseline: 16.2213x; 1.0072x over previous
import jax
import jax.numpy as jnp
from jax import lax
from jax.experimental import pallas as pl
from jax.experimental.pallas import tpu as pltpu
from jax.experimental.pallas import tpu_sc as plsc

N = 10000
E = 320000
D = 128
DH = D // 2

NC = 2
NS = 16
CPT = E // NS
CHUNK = 80
NCHUNK = CPT // CHUNK
HCHUNK = NCHUNK // 2
NP = 10240
RPT = NP // NS
ZR = 128


NB = 5


def _sc_body(xv_hbm, eidx_hbm, out_s, out_c,
             sidx_all, didx_all, rows, ones16, zrows, z16,
             isem, gsem, ssem, csem, zsem, acc, cnt):
    c = lax.axis_index("c")
    s = lax.axis_index("s")
    jbase = s * NCHUNK
    rbase = s * RPT

    zero16 = jnp.zeros((16,), jnp.float32)
    one16 = jnp.ones((16,), jnp.float32)

    idx_in = pltpu.async_copy(
        eidx_hbm.at[pl.ds(jbase, NCHUNK)], sidx_all, isem)
    idx_in2 = pltpu.async_copy(
        eidx_hbm.at[pl.ds(E // CHUNK + jbase, NCHUNK)], didx_all, isem)

    def fill_z(i, carry):
        for k in range(DH // 16):
            zrows[i, pl.ds(k * 16, 16)] = zero16
        z16[i, :] = zero16
        return carry

    lax.fori_loop(0, ZR, fill_z, 0)

    def fill_one(i, carry):
        ones16[i, :] = one16
        return carry

    lax.fori_loop(0, CHUNK, fill_one, 0)

    def zcopy(k, carry):
        r0 = rbase + k * ZR
        pltpu.async_copy(zrows, acc.at[pl.ds(r0, ZR)], zsem)
        pltpu.async_copy(z16, cnt.at[pl.ds(r0, ZR)], zsem)
        return carry

    lax.fori_loop(0, RPT // ZR, zcopy, 0)
    idx_in.wait()
    idx_in2.wait()

    cvec = jnp.full((16,), c, jnp.int32)

    def xform(r, carry):
        for g in range(CHUNK // 16):
            v = sidx_all[r, pl.ds(g * 16, 16)]
            sidx_all[r, pl.ds(g * 16, 16)] = v + v + cvec
        return carry

    lax.fori_loop(0, NCHUNK, xform, 0)

    def zdrain(k, carry):
        r0 = rbase + k * ZR
        pltpu.make_async_copy(zrows, acc.at[pl.ds(r0, ZR)], zsem).wait()
        pltpu.make_async_copy(z16, cnt.at[pl.ds(r0, ZR)], zsem).wait()
        return carry

    lax.fori_loop(0, RPT // ZR, zdrain, 0)
    plsc.subcore_barrier()

    def start_g(j, b):
        pltpu.async_copy(xv_hbm.at[sidx_all.at[j]], rows.at[b], gsem.at[b])

    def wait_g(j, b):
        pltpu.make_async_copy(
            xv_hbm.at[sidx_all.at[j]], rows.at[b], gsem.at[b]).wait()

    def start_s(j, b):
        pltpu.async_copy(rows.at[b], acc.at[didx_all.at[j]], ssem.at[b],
                         add=True)

        @pl.when((j < HCHUNK) == (c == 0))
        def _():
            pltpu.async_copy(ones16, cnt.at[didx_all.at[j]], csem, add=True)

    def wait_s(j, b):
        pltpu.make_async_copy(
            rows.at[b], acc.at[didx_all.at[j]], ssem.at[b]).wait()

    for b in range(NB):
        start_g(b, b)

    def edge(k, carry):
        j = NB * k
        for b in range(NB):
            wait_g(j + b, b)
            start_s(j + b, b)
        for b in range(NB):
            wait_s(j + b, b)
            start_g(j + NB + b, b)
        return carry

    lax.fori_loop(0, NCHUNK // NB - 1, edge, 0)
    jlast = NCHUNK - NB
    for b in range(NB):
        wait_g(jlast + b, b)
        start_s(jlast + b, b)
    for b in range(NB):
        wait_s(jlast + b, b)

    def drain(k, carry):
        j = k + c * HCHUNK
        pltpu.make_async_copy(ones16, cnt.at[didx_all.at[j]], csem).wait()
        return carry

    lax.fori_loop(0, HCHUNK, drain, 0)

    plsc.subcore_barrier()

    OC = RPT // CHUNK

    def oc_in(k, b):
        pltpu.async_copy(acc.at[pl.ds(rbase + k * CHUNK, CHUNK)],
                         rows.at[b], gsem.at[b])

    def oc_wait(k, b):
        pltpu.make_async_copy(acc.at[pl.ds(rbase + k * CHUNK, CHUNK)],
                              rows.at[b], gsem.at[b]).wait()

    for b in range(NB):
        oc_in(b, b)
    for k in range(OC):
        b = k % NB
        oc_wait(k, b)
        pltpu.sync_copy(rows.at[b],
                        out_s.at[pl.ds(rbase + k * CHUNK, CHUNK),
                                 pl.ds(c * DH, DH)])
        if k + NB < OC:
            oc_in(k + NB, b)

    for k in range(RPT // ZR):
        r0 = rbase + k * ZR
        pltpu.sync_copy(cnt.at[pl.ds(r0, ZR)], z16)
        pltpu.sync_copy(z16, out_c.at[c, pl.ds(r0, ZR)])


_sc_segment_sum = pl.kernel(
    _sc_body,
    out_type=(
        jax.ShapeDtypeStruct((NP, D), jnp.float32),
        jax.ShapeDtypeStruct((NC, NP, 16), jnp.float32),
    ),
    mesh=plsc.VectorSubcoreMesh(
        core_axis_name="c", subcore_axis_name="s",
        num_cores=NC, num_subcores=NS),
    compiler_params=pltpu.CompilerParams(use_tc_tiling_on_sc=False),
    scratch_types=[
        pltpu.VMEM((NCHUNK, CHUNK), jnp.int32),
        pltpu.VMEM((NCHUNK, CHUNK), jnp.int32),
        pltpu.VMEM((NB, CHUNK, DH), jnp.float32),
        pltpu.VMEM((CHUNK, 16), jnp.float32),
        pltpu.VMEM((ZR, DH), jnp.float32),
        pltpu.VMEM((ZR, 16), jnp.float32),
        pltpu.SemaphoreType.DMA,
        pltpu.SemaphoreType.DMA((NB,)),
        pltpu.SemaphoreType.DMA((NB,)),
        pltpu.SemaphoreType.DMA,
        pltpu.SemaphoreType.DMA,
        pltpu.VMEM_SHARED((NP, DH), jnp.float32),
        pltpu.VMEM_SHARED((NP, 16), jnp.float32),
    ],
)


R = 1024
RB = R // 8


def _tc_body(s_ref, c0_ref, c1_ref, x_ref, wl_ref, wr_ref, b_ref, o_ref):
    ssum = s_ref[...]
    c8 = c0_ref[...] + c1_ref[...]
    rep = jnp.repeat(c8, 8, axis=0)
    p_row = lax.broadcasted_iota(jnp.int32, (R, D), 0)
    q_col = lax.broadcasted_iota(jnp.int32, (R, D), 1)
    sel = (q_col // 16) == (p_row % 8)
    cnt = jnp.sum(jnp.where(sel, rep, 0.0), axis=-1,
                  keepdims=True) * (1.0 / 16.0)
    mean = ssum / jnp.clip(cnt, 1.0, None)
    dn = (((1,), (1,)), ((), ()))
    out = (lax.dot_general(mean, wl_ref[...], dn,
                           preferred_element_type=jnp.float32)
           + lax.dot_general(x_ref[...], wr_ref[...], dn,
                             preferred_element_type=jnp.float32)
           + b_ref[...])
    nrm = jnp.sqrt(jnp.sum(out * out, axis=-1, keepdims=True))
    o_ref[...] = out / jnp.maximum(nrm, 1e-12)


_CB = NP // 8 // RB


def _tc_combine(ps, pcv, x, W_l, W_r, b2):
    return pl.pallas_call(
        _tc_body,
        grid=(N // R + 1,),
        in_specs=[
            pl.BlockSpec((R, D), lambda i: (i, 0)),
            pl.BlockSpec((RB, D), lambda i: (i, 0)),
            pl.BlockSpec((RB, D), lambda i: (i + _CB, 0)),
            pl.BlockSpec((R, D), lambda i: (i, 0)),
            pl.BlockSpec((D, D), lambda i: (0, 0)),
            pl.BlockSpec((D, D), lambda i: (0, 0)),
            pl.BlockSpec((1, D), lambda i: (0, 0)),
        ],
        out_specs=pl.BlockSpec((R, D), lambda i: (i, 0)),
        out_shape=jax.ShapeDtypeStruct((N, D), jnp.float32),
    )(ps, pcv, pcv, x, W_l, W_r, b2)


@jax.jit
def kernel(x, edge_index, W_l, W_r, b_l):
    e2 = edge_index.reshape(2 * (E // CHUNK), CHUNK)
    xv = x.reshape(2 * N, DH)
    ps, pc = _sc_segment_sum(xv, e2)
    pcv = pc.reshape(NC * NP // 8, D)
    return _tc_combine(ps, pcv, x, W_l, W_r, b_l.reshape(1, D))
